# baseline (speedup 1.0000x reference)
import dataclasses
import functools

import jax
import jax.numpy as jnp
from jax import lax
from jax.experimental import pallas as pl
from jax.experimental.pallas import tpu as pltpu
from jax.experimental.pallas import tpu_sc as plsc

TOPK = 8
NEXP = 64
BLK = 1024
HBLK = BLK // 2

NTOK = 8192
NCHUNK = 2
CTOK = NTOK // NCHUNK
NBLK = CTOK // BLK
NC, NS, L = 2, 16, 16
NW = NC * NS
WPB = NW // NBLK
PPW = HBLK // WPB


def _scores_block(x_ref, wt_ref, e_ref):
    x = x_ref[...]
    wt = wt_ref[...]
    logits = jax.lax.dot_general(
        x, wt, (((1,), (0,)), ((), ())),
        preferred_element_type=jnp.float32,
        precision=jax.lax.Precision.DEFAULT,
    )
    m = jnp.max(logits, axis=1, keepdims=True)
    e = jnp.exp(logits - m)
    e_ref[...] = jnp.concatenate(
        [lax.slice(e, (0, 0), (HBLK, NEXP)),
         lax.slice(e, (HBLK, 0), (BLK, NEXP))], axis=1)


def _tc_scores(x, wt, chunk):
    nb = NBLK
    return pl.pallas_call(
        _scores_block,
        grid=(nb,),
        in_specs=[
            pl.BlockSpec((BLK, x.shape[1]),
                         lambda i, c=chunk: (c * NBLK + i, 0)),
            pl.BlockSpec((x.shape[1], NEXP), lambda i: (0, 0)),
        ],
        out_specs=pl.BlockSpec((HBLK, 2 * NEXP), lambda i: (i, 0)),
        out_shape=jax.ShapeDtypeStruct((nb * HBLK, 2 * NEXP), jnp.float32),
        compiler_params=pltpu.CompilerParams(
            dimension_semantics=("parallel",),
        ),
    )(x, wt)


def _token_topk(sc_v, off, lane, mask8):
    ks = []
    vs = []
    for j in range(NEXP // L):
        kj = sc_v[pl.ds(off + j * L, L)]
        skj, svj = plsc.sort_key_val(
            kj, lane + j * L, descending=(j % 2 == 0))
        ks.append(skj)
        vs.append(svj)
    k01, v01 = plsc.sort_key_val(
        jnp.where(mask8, ks[0], ks[1]),
        jnp.where(mask8, vs[0], vs[1]), descending=True)
    k23, v23 = plsc.sort_key_val(
        jnp.where(mask8, ks[2], ks[3]),
        jnp.where(mask8, vs[2], vs[3]), descending=False)
    kf, vf = plsc.sort_key_val(
        jnp.where(mask8, k01, k23),
        jnp.where(mask8, v01, v23), descending=True)
    km = jnp.where(mask8, kf, 0.0)
    return km / jnp.sum(km), vf


def _sc_topk_body(scores_hbm, w_hbm, i_hbm, sc_v, wv, iv, sem):
    wid = lax.axis_index("s") * NC + lax.axis_index("c")
    blk = wid // WPB
    r0 = (wid % WPB) * PPW
    row0 = blk * HBLK + r0
    pltpu.async_copy(
        scores_hbm.at[pl.ds(row0 * 2 * NEXP, PPW * 2 * NEXP)], sc_v, sem
    ).wait()

    lane = lax.iota(jnp.int32, L)
    mask8 = lane < TOPK

    @pl.loop(0, PPW)
    def _(p):
        wa, ia = _token_topk(sc_v, p * 2 * NEXP, lane, mask8)
        wb, ib = _token_topk(sc_v, p * 2 * NEXP + NEXP, lane, mask8)
        plsc.store_compressed(wv.at[pl.ds(p * TOPK, L)], wa, mask=mask8)
        plsc.store_compressed(iv.at[pl.ds(p * TOPK, L)], ia, mask=mask8)
        plsc.store_compressed(
            wv.at[pl.ds((PPW + p) * TOPK, L)], wb, mask=mask8)
        plsc.store_compressed(
            iv.at[pl.ds((PPW + p) * TOPK, L)], ib, mask=mask8)

    tok_a = blk * BLK + r0
    for half, tok in ((0, tok_a), (1, tok_a + HBLK)):
        pltpu.async_copy(
            wv.at[pl.ds(half * PPW * TOPK, PPW * TOPK)],
            w_hbm.at[pl.ds(tok * TOPK, PPW * TOPK)], sem
        ).wait()
        pltpu.async_copy(
            iv.at[pl.ds(half * PPW * TOPK, PPW * TOPK)],
            i_hbm.at[pl.ds(tok * TOPK, PPW * TOPK)], sem
        ).wait()


_sc_compiler_params = pltpu.CompilerParams()
if "needs_layout_passes" in pltpu.CompilerParams.__dataclass_fields__:
    _sc_compiler_params = dataclasses.replace(
        _sc_compiler_params, needs_layout_passes=False)

_sc_topk = functools.partial(
    pl.kernel,
    mesh=plsc.VectorSubcoreMesh(core_axis_name="c", subcore_axis_name="s"),
    compiler_params=_sc_compiler_params,
    out_type=(
        jax.ShapeDtypeStruct((CTOK * TOPK,), jnp.float32),
        jax.ShapeDtypeStruct((CTOK * TOPK,), jnp.int32),
    ),
    scratch_types=[
        pltpu.VMEM((PPW * 2 * NEXP,), jnp.float32),
        pltpu.VMEM((2 * PPW * TOPK + L,), jnp.float32),
        pltpu.VMEM((2 * PPW * TOPK + L,), jnp.int32),
        pltpu.SemaphoreType.DMA,
    ],
)(_sc_topk_body)


@jax.jit
def kernel(hidden_states, W):
    b, s, h = hidden_states.shape
    n = b * s
    x = hidden_states.reshape(n, h)
    wt = W.astype(jnp.float32).T
    es = [_tc_scores(x, wt, c) for c in range(NCHUNK)]
    ws = []
    idxs = []
    for e in es:
        wc, ic = _sc_topk(e.reshape(-1))
        ws.append(wc.reshape(CTOK, TOPK))
        idxs.append(ic.reshape(CTOK, TOPK))
    return jnp.concatenate(ws, axis=0), jnp.concatenate(idxs, axis=0)


# speedup vs baseline: 1.3959x; 1.3959x over previous
import functools

import jax
import jax.numpy as jnp
from jax.experimental import pallas as pl
from jax.experimental.pallas import tpu as pltpu

TOPK = 8
NEXP = 64
BLK = 1024


CHUNK = 128


def _topk_chunk(e):
    colid = jax.lax.broadcasted_iota(jnp.int32, e.shape, 1).astype(jnp.float32)
    cur = e
    vals = []
    idxs = []
    for _ in range(TOPK):
        mk = jnp.max(cur, axis=1, keepdims=True)
        ik = jnp.min(jnp.where(cur == mk, colid, float(NEXP)), axis=1,
                     keepdims=True)
        vals.append(mk)
        idxs.append(ik)
        cur = jnp.where(colid == ik, -1.0, cur)
    w = jnp.concatenate(vals, axis=1)
    w = w / (jnp.sum(w, axis=1, keepdims=True) + 1e-20)
    return w, jnp.concatenate(idxs, axis=1).astype(jnp.int32)


def _gate_block(x_ref, wt_ref, w_out_ref, i_out_ref):
    x = x_ref[...]
    wt = wt_ref[...]
    logits = jax.lax.dot_general(
        x, wt, (((1,), (0,)), ((), ())),
        preferred_element_type=jnp.float32,
        precision=jax.lax.Precision.DEFAULT,
    )
    m = jnp.max(logits, axis=1, keepdims=True)
    e = jnp.exp(logits - m)
    for c in range(BLK // CHUNK):
        lo, hi = c * CHUNK, (c + 1) * CHUNK
        w, ik = _topk_chunk(jax.lax.slice(e, (lo, 0), (hi, NEXP)))
        w_out_ref[lo:hi, :] = w
        i_out_ref[lo:hi, :] = ik


@jax.jit
def kernel(hidden_states, W):
    b, s, h = hidden_states.shape
    n = b * s
    x = hidden_states.reshape(n, h)
    wt = W.astype(jnp.float32).T
    grid = (n // BLK,)
    w_out, i_out = pl.pallas_call(
        _gate_block,
        grid=grid,
        in_specs=[
            pl.BlockSpec((BLK, h), lambda i: (i, 0)),
            pl.BlockSpec((h, NEXP), lambda i: (0, 0)),
        ],
        out_specs=[
            pl.BlockSpec((BLK, TOPK), lambda i: (i, 0)),
            pl.BlockSpec((BLK, TOPK), lambda i: (i, 0)),
        ],
        out_shape=[
            jax.ShapeDtypeStruct((n, TOPK), jnp.float32),
            jax.ShapeDtypeStruct((n, TOPK), jnp.int32),
        ],
        compiler_params=pltpu.CompilerParams(
            dimension_semantics=("parallel",),
        ),
    )(x, wt)
    return w_out, i_out
